# baseline (speedup 1.0000x reference)
import functools

import jax
import jax.numpy as jnp
from jax.experimental import pallas as pl

N_TOK = 16384
N_EMBD = 2048
N_EXPERTS = 16
TOP_K = 2

TILE = 512


def _router_krn(x_ref, w_ref, b_ref, epst_ref, routt_ref, idxt_ref, g1t_ref):
    x = x_ref[...]
    y = jnp.dot(x, w_ref[...], preferred_element_type=jnp.float32)
    yt = y.T + b_ref[...]
    logits = yt[:N_EXPERTS, :]
    nlog = yt[N_EXPERTS:, :]
    noisy = logits + epst_ref[...] * jax.nn.softplus(nlog)

    m1 = jnp.max(noisy, axis=0, keepdims=True)
    e_all = jnp.exp(noisy - m1)
    g1t_ref[...] = e_all / jnp.sum(e_all, axis=0, keepdims=True)

    lane = jax.lax.broadcasted_iota(jnp.int32, noisy.shape, 0)
    big = jnp.int32(N_EXPERTS)
    i1 = jnp.min(jnp.where(noisy == m1, lane, big), axis=0, keepdims=True)
    rest = jnp.where(lane == i1, -jnp.inf, noisy)
    m2 = jnp.max(rest, axis=0, keepdims=True)
    i2 = jnp.min(jnp.where(rest == m2, lane, big), axis=0, keepdims=True)
    idxt_ref[...] = jnp.concatenate([i1, i2], axis=0)

    keep = (lane == i1) | (lane == i2)
    routt_ref[...] = jnp.where(keep, e_all, 0.0) / (1.0 + jnp.exp(m2 - m1))


@functools.partial(jax.jit, static_argnames=())
def kernel(mh_output, W_route, b_route, W_noise, b_noise, noise_eps):
    grid = (N_TOK // TILE,)
    W = jnp.concatenate([W_route, W_noise], axis=1)
    b = jnp.concatenate([b_route, b_noise]).reshape(2 * N_EXPERTS, 1)
    epst = noise_eps.T
    routt, idxt, g1t = pl.pallas_call(
        _router_krn,
        grid=grid,
        in_specs=[
            pl.BlockSpec((TILE, N_EMBD), lambda i: (i, 0)),
            pl.BlockSpec((N_EMBD, 2 * N_EXPERTS), lambda i: (0, 0)),
            pl.BlockSpec((2 * N_EXPERTS, 1), lambda i: (0, 0)),
            pl.BlockSpec((N_EXPERTS, TILE), lambda i: (0, i)),
        ],
        out_specs=[
            pl.BlockSpec((N_EXPERTS, TILE), lambda i: (0, i)),
            pl.BlockSpec((TOP_K, TILE), lambda i: (0, i)),
            pl.BlockSpec((N_EXPERTS, TILE), lambda i: (0, i)),
        ],
        out_shape=[
            jax.ShapeDtypeStruct((N_EXPERTS, N_TOK), jnp.float32),
            jax.ShapeDtypeStruct((TOP_K, N_TOK), jnp.int32),
            jax.ShapeDtypeStruct((N_EXPERTS, N_TOK), jnp.float32),
        ],
    )(mh_output, W, b, epst)
    return (routt.T, idxt.T, g1t.T)


# speedup vs baseline: 1.1948x; 1.1948x over previous
import functools

import jax
import jax.numpy as jnp
from jax.experimental import pallas as pl
from jax.experimental.pallas import tpu as pltpu

N_TOK = 16384
N_EMBD = 2048
N_EXPERTS = 16
TOP_K = 2

TILE = 1024


def _router_krn(x_ref, w_ref, b_ref, epst_ref, routt_ref, idxt_ref, g1t_ref):
    x = x_ref[...]
    y = jnp.dot(x, w_ref[...], preferred_element_type=jnp.float32)
    yt = y.T + b_ref[...]
    logits = yt[:N_EXPERTS, :]
    nlog = yt[N_EXPERTS:, :]
    noisy = logits + epst_ref[...] * jax.nn.softplus(nlog)

    m1 = jnp.max(noisy, axis=0, keepdims=True)
    e_all = jnp.exp(noisy - m1)
    g1t_ref[...] = e_all / jnp.sum(e_all, axis=0, keepdims=True)

    lane = jax.lax.broadcasted_iota(jnp.int32, noisy.shape, 0)
    big = jnp.int32(N_EXPERTS)
    i1 = jnp.min(jnp.where(noisy == m1, lane, big), axis=0, keepdims=True)
    rest = jnp.where(lane == i1, -jnp.inf, noisy)
    m2 = jnp.max(rest, axis=0, keepdims=True)
    i2 = jnp.min(jnp.where(rest == m2, lane, big), axis=0, keepdims=True)
    idxt_ref[...] = jnp.concatenate([i1, i2], axis=0)

    keep = (lane == i1) | (lane == i2)
    routt_ref[...] = jnp.where(keep, e_all, 0.0) / (1.0 + jnp.exp(m2 - m1))


@functools.partial(jax.jit, static_argnames=())
def kernel(mh_output, W_route, b_route, W_noise, b_noise, noise_eps):
    grid = (N_TOK // TILE,)
    W = jnp.concatenate([W_route, W_noise], axis=1)
    b = jnp.concatenate([b_route, b_noise]).reshape(2 * N_EXPERTS, 1)
    epst = noise_eps.T
    routt, idxt, g1t = pl.pallas_call(
        _router_krn,
        grid=grid,
        compiler_params=pltpu.CompilerParams(
            dimension_semantics=("parallel",)),
        in_specs=[
            pl.BlockSpec((TILE, N_EMBD), lambda i: (i, 0)),
            pl.BlockSpec((N_EMBD, 2 * N_EXPERTS), lambda i: (0, 0)),
            pl.BlockSpec((2 * N_EXPERTS, 1), lambda i: (0, 0)),
            pl.BlockSpec((N_EXPERTS, TILE), lambda i: (0, i)),
        ],
        out_specs=[
            pl.BlockSpec((N_EXPERTS, TILE), lambda i: (0, i)),
            pl.BlockSpec((TOP_K, TILE), lambda i: (0, i)),
            pl.BlockSpec((N_EXPERTS, TILE), lambda i: (0, i)),
        ],
        out_shape=[
            jax.ShapeDtypeStruct((N_EXPERTS, N_TOK), jnp.float32),
            jax.ShapeDtypeStruct((TOP_K, N_TOK), jnp.int32),
            jax.ShapeDtypeStruct((N_EXPERTS, N_TOK), jnp.float32),
        ],
    )(mh_output, W, b, epst)
    return (routt.T, idxt.T, g1t.T)
